# baseline (speedup 1.0000x reference)
import jax
import jax.numpy as jnp
from jax import lax
from jax.experimental import pallas as pl
from jax.experimental.pallas import tpu as pltpu
from jax.experimental.pallas import tpu_sc as plsc

_NC = 2
_NS = 16
_NW = _NC * _NS
_B = 128
_G = 32

_NBUF = 4
_DEP = 2
_DWIN = 8

def _sc_mesh():
    return plsc.VectorSubcoreMesh(core_axis_name="c", subcore_axis_name="s",
                                  num_cores=_NC, num_subcores=_NS)


_SC_PARAMS = pltpu.CompilerParams(use_tc_tiling_on_sc=False)


def _deg_call(eidx, ones8, zeros8, npad, nb, r):
    rpt = npad // _NS
    kv = nb + (1 if r else 0)

    def body(eidx_hbm, ones_hbm, zeros_hbm, out_hbm, acc_sh, didx_v, ones_v,
             sem, zsem):
        c = lax.axis_index("c")
        s = lax.axis_index("s")
        w = c * _NS + s
        base = nb * w + jnp.minimum(w, r)
        z = pltpu.async_copy(zeros_hbm, acc_sh.at[pl.ds(s * rpt, rpt)], zsem)
        o = pltpu.async_copy(ones_hbm, ones_v, sem)
        i = pltpu.async_copy(eidx_hbm.at[1, pl.ds(base, nb)],
                             didx_v.at[pl.ds(0, nb)], zsem)
        z.wait()
        o.wait()
        i.wait()
        if r:
            pl.when(w < r)(lambda: pltpu.sync_copy(
                eidx_hbm.at[1, pl.ds(base + nb, 1)], didx_v.at[pl.ds(nb, 1)]))
        plsc.subcore_barrier()
        descs = [None] * nb
        for j in range(nb):
            if j >= _DWIN:
                descs[j - _DWIN].wait()
            descs[j] = pltpu.async_copy(ones_v, acc_sh.at[didx_v.at[j]], sem,
                                        add=True)
        for j in range(max(0, nb - _DWIN), nb):
            descs[j].wait()
        if r:
            pl.when(w < r)(lambda: pltpu.sync_copy(
                ones_v, acc_sh.at[didx_v.at[nb]], add=True))
        plsc.subcore_barrier()
        pltpu.sync_copy(acc_sh.at[pl.ds(s * rpt, rpt)],
                        out_hbm.at[c, pl.ds(s * rpt, rpt)])

    f = pl.kernel(
        body,
        out_type=jax.ShapeDtypeStruct((_NC, npad, 8), jnp.float32),
        mesh=_sc_mesh(),
        scratch_types=[
            pltpu.VMEM_SHARED((npad, 8), jnp.float32),
            pltpu.VMEM((kv, _B), jnp.int32),
            pltpu.VMEM((_B, 8), jnp.float32),
            pltpu.SemaphoreType.DMA,
            pltpu.SemaphoreType.DMA,
        ],
        compiler_params=_SC_PARAMS,
    )
    return f(eidx, ones8, zeros8)


def _agg_call(table, eidx, zerosh, npad, nb, r, h):
    rpt = npad // _NS
    kv = nb + (1 if r else 0)

    def body(tab_hbm, eidx_hbm, zeros_hbm, out_hbm,
             acc_sh, sidx_v, didx_v, *rest):
        rows = rest[:_NBUF]
        gsem = rest[_NBUF:2 * _NBUF]
        ssem = rest[2 * _NBUF:3 * _NBUF]
        zsem = rest[3 * _NBUF]
        c = lax.axis_index("c")
        s = lax.axis_index("s")
        w = c * _NS + s
        base = nb * w + jnp.minimum(w, r)
        z = pltpu.async_copy(zeros_hbm, acc_sh.at[pl.ds(s * rpt, rpt)], zsem)
        a = pltpu.async_copy(eidx_hbm.at[0, pl.ds(base, nb)],
                             sidx_v.at[pl.ds(0, nb)], gsem[0])
        b = pltpu.async_copy(eidx_hbm.at[1, pl.ds(base, nb)],
                             didx_v.at[pl.ds(0, nb)], gsem[1])
        z.wait()
        a.wait()
        b.wait()
        if r:
            def tail_stage():
                pltpu.sync_copy(eidx_hbm.at[0, pl.ds(base + nb, 1)],
                                sidx_v.at[pl.ds(nb, 1)])
                pltpu.sync_copy(eidx_hbm.at[1, pl.ds(base + nb, 1)],
                                didx_v.at[pl.ds(nb, 1)])
            pl.when(w < r)(tail_stage)
        plsc.subcore_barrier()

        def pipe(lo, hi):
            gd = [None] * _NBUF
            sd = [None] * _NBUF
            n_ = hi - lo
            for t in range(n_ + _DEP):
                if t < n_:
                    bi = t % _NBUF
                    if t >= _NBUF:
                        sd[bi].wait()
                    gd[bi] = pltpu.async_copy(
                        tab_hbm.at[sidx_v.at[lo + t]], rows[bi], gsem[bi])
                if t >= _DEP:
                    j = t - _DEP
                    bj = j % _NBUF
                    gd[bj].wait()
                    sd[bj] = pltpu.async_copy(
                        rows[bj], acc_sh.at[didx_v.at[lo + j]], ssem[bj],
                        add=True)
            for j in range(max(0, n_ - _NBUF), n_):
                sd[j % _NBUF].wait()

        pipe(0, nb)
        if r:
            def tail_chunk():
                pltpu.async_copy(tab_hbm.at[sidx_v.at[nb]], rows[0],
                                 gsem[0]).wait()
                pltpu.async_copy(rows[0], acc_sh.at[didx_v.at[nb]], ssem[0],
                                 add=True).wait()
            pl.when(w < r)(tail_chunk)
        plsc.subcore_barrier()
        pltpu.sync_copy(acc_sh.at[pl.ds(s * rpt, rpt)],
                        out_hbm.at[c, pl.ds(s * rpt, rpt)])

    f = pl.kernel(
        body,
        out_type=jax.ShapeDtypeStruct((_NC, npad, h), jnp.bfloat16),
        mesh=_sc_mesh(),
        scratch_types=(
            [pltpu.VMEM_SHARED((npad, h), jnp.bfloat16),
             pltpu.VMEM((kv, _B), jnp.int32),
             pltpu.VMEM((kv, _B), jnp.int32)]
            + [pltpu.VMEM((_B, h), jnp.bfloat16)] * _NBUF
            + [pltpu.SemaphoreType.DMA] * (2 * _NBUF + 1)
        ),
        compiler_params=_SC_PARAMS,
    )
    return f(table, eidx, zerosh)


def _dinv_of(degp_ref):
    deg = degp_ref[0, :, 0:1] + degp_ref[1, :, 0:1]
    return lax.rsqrt(deg + 1.0)


def _tc1(x, W1, degp, n, npad, h):
    def body(x_hbm, w1_ref, degp_ref, out_ref, x_vmem, sem):
        pltpu.async_copy(x_hbm, x_vmem, sem).wait()
        t1 = jnp.dot(x_vmem[...], w1_ref[...],
                     preferred_element_type=jnp.float32)
        dinv = _dinv_of(degp_ref)
        out_ref[pl.ds(0, n), :] = (t1 * dinv[:n]).astype(jnp.bfloat16)
        out_ref[pl.ds(n, npad - n), :] = jnp.zeros((npad - n, h), jnp.bfloat16)

    return pl.pallas_call(
        body, out_shape=jax.ShapeDtypeStruct((npad, h), jnp.bfloat16),
        in_specs=[pl.BlockSpec(memory_space=pltpu.MemorySpace.HBM),
                  pl.BlockSpec(memory_space=pltpu.MemorySpace.VMEM),
                  pl.BlockSpec(memory_space=pltpu.MemorySpace.VMEM)],
        scratch_shapes=[pltpu.VMEM((n, x.shape[1]), jnp.float32),
                        pltpu.SemaphoreType.DMA],
    )(x, W1, degp)


def _tc2(e1, h1p, degp, b1, npad, h):
    def body(e_ref, h1p_ref, degp_ref, b1_ref, out_ref):
        dinv = _dinv_of(degp_ref)
        agg = (e_ref[0].astype(jnp.float32) + e_ref[1].astype(jnp.float32)
               + h1p_ref[...].astype(jnp.float32)) * dinv
        h1 = jnp.maximum(agg + b1_ref[...], 0.0)
        out_ref[...] = (h1 * dinv).astype(jnp.bfloat16)

    return pl.pallas_call(
        body, out_shape=jax.ShapeDtypeStruct((npad, h), jnp.bfloat16),
    )(e1, h1p, degp, b1)


def _tc3(e2, v2, degp, W2, b2, batch2d, fc1_w, fc1_b, fc2_w, fc2_b, npad, ncls):
    def body(e_ref, v2_ref, degp_ref, w2_ref, b2_ref, batch_ref,
             fc1w_ref, fc1b_ref, fc2w_ref, fc2b_ref, out_ref):
        dinv = _dinv_of(degp_ref)
        a2 = (e_ref[0].astype(jnp.float32) + e_ref[1].astype(jnp.float32)
              + v2_ref[...].astype(jnp.float32)) * dinv
        h2 = jnp.maximum(
            jnp.dot(a2, w2_ref[...], preferred_element_type=jnp.float32)
            + b2_ref[...], 0.0)
        g = lax.broadcasted_iota(jnp.int32, (_G, 1), 0)
        mask = (batch_ref[...] == g).astype(jnp.float32)
        sums = jnp.dot(mask, h2, preferred_element_type=jnp.float32)
        counts = jnp.sum(mask, axis=1, keepdims=True)
        pooled = sums / jnp.maximum(counts, 1.0)
        z1 = jnp.maximum(
            jnp.dot(pooled, fc1w_ref[...], preferred_element_type=jnp.float32)
            + fc1b_ref[...], 0.0)
        out_ref[...] = (
            jnp.dot(z1, fc2w_ref[...], preferred_element_type=jnp.float32)
            + fc2b_ref[...])

    return pl.pallas_call(
        body, out_shape=jax.ShapeDtypeStruct((_G, ncls), jnp.float32),
    )(e2, v2, degp, W2, b2, batch2d, fc1_w, fc1_b, fc2_w, fc2_b)


def kernel(x, edge_index, batch, W1, b1, W2, b2, fc1_w, fc1_b, fc2_w, fc2_b):
    n, d_in = x.shape
    h = W1.shape[1]
    h2 = W2.shape[1]
    ncls = fc2_w.shape[1]
    e = edge_index.shape[1]

    nt = -(-e // _B)
    nb = nt // _NW
    r = nt % _NW
    rpt = -(-(n + 1) // _NS)
    rpt += (-rpt) % 8
    npad = rpt * _NS

    ei = edge_index.astype(jnp.int32)
    if e % _B:
        fill = jnp.full((2, nt * _B - e), n, jnp.int32)
        ei = jnp.concatenate([ei, fill], axis=1)
    eidx = ei.reshape(2, nt, _B)

    batch2d = jnp.pad(batch.astype(jnp.int32), (0, npad - n),
                      constant_values=_G).reshape(1, npad)

    ones8 = jnp.ones((_B, 8), jnp.float32)
    zeros8 = jnp.zeros((rpt, 8), jnp.float32)
    zerosh = jnp.zeros((rpt, h), jnp.bfloat16)

    degp = _deg_call(eidx, ones8, zeros8, npad, nb, r)
    h1p = _tc1(x, W1, degp, n, npad, h)
    e1 = _agg_call(h1p, eidx, zerosh, npad, nb, r, h)
    v2 = _tc2(e1, h1p, degp, b1.reshape(1, h), npad, h)
    e2 = _agg_call(v2, eidx, zerosh, npad, nb, r, h)
    out = _tc3(e2, v2, degp, W2, b2.reshape(1, h2), batch2d,
               fc1_w, fc1_b.reshape(1, h), fc2_w, fc2_b.reshape(1, ncls),
               npad, ncls)
    return out


# speedup vs baseline: 1.0562x; 1.0562x over previous
import jax
import jax.numpy as jnp
from jax import lax
from jax.experimental import pallas as pl
from jax.experimental.pallas import tpu as pltpu
from jax.experimental.pallas import tpu_sc as plsc

_NC = 2
_NS = 16
_NW = _NC * _NS
_B = 128
_G = 32

_NBUF = 8
_DEP = 6
_DWIN = 8

def _sc_mesh():
    return plsc.VectorSubcoreMesh(core_axis_name="c", subcore_axis_name="s",
                                  num_cores=_NC, num_subcores=_NS)


_SC_PARAMS = pltpu.CompilerParams(use_tc_tiling_on_sc=False)


def _deg_call(eidx, ones8, zeros8, npad, nb, r):
    rpt = npad // _NS
    kv = nb + (1 if r else 0)

    def body(eidx_hbm, ones_hbm, zeros_hbm, out_hbm, acc_sh, didx_v, ones_v,
             sem, zsem):
        c = lax.axis_index("c")
        s = lax.axis_index("s")
        w = c * _NS + s
        base = nb * w + jnp.minimum(w, r)
        z = pltpu.async_copy(zeros_hbm, acc_sh.at[pl.ds(s * rpt, rpt)], zsem)
        o = pltpu.async_copy(ones_hbm, ones_v, sem)
        i = pltpu.async_copy(eidx_hbm.at[1, pl.ds(base, nb)],
                             didx_v.at[pl.ds(0, nb)], zsem)
        z.wait()
        o.wait()
        i.wait()
        if r:
            pl.when(w < r)(lambda: pltpu.sync_copy(
                eidx_hbm.at[1, pl.ds(base + nb, 1)], didx_v.at[pl.ds(nb, 1)]))
        plsc.subcore_barrier()
        descs = [None] * nb
        for j in range(nb):
            if j >= _DWIN:
                descs[j - _DWIN].wait()
            descs[j] = pltpu.async_copy(ones_v, acc_sh.at[didx_v.at[j]], sem,
                                        add=True)
        for j in range(max(0, nb - _DWIN), nb):
            descs[j].wait()
        if r:
            pl.when(w < r)(lambda: pltpu.sync_copy(
                ones_v, acc_sh.at[didx_v.at[nb]], add=True))
        plsc.subcore_barrier()
        pltpu.sync_copy(acc_sh.at[pl.ds(s * rpt, rpt)],
                        out_hbm.at[c, pl.ds(s * rpt, rpt)])

    f = pl.kernel(
        body,
        out_type=jax.ShapeDtypeStruct((_NC, npad, 8), jnp.float32),
        mesh=_sc_mesh(),
        scratch_types=[
            pltpu.VMEM_SHARED((npad, 8), jnp.float32),
            pltpu.VMEM((kv, _B), jnp.int32),
            pltpu.VMEM((_B, 8), jnp.float32),
            pltpu.SemaphoreType.DMA,
            pltpu.SemaphoreType.DMA,
        ],
        compiler_params=_SC_PARAMS,
    )
    return f(eidx, ones8, zeros8)


def _agg_call(table, eidx, zerosh, npad, nb, r, h):
    rpt = npad // _NS
    kv = nb + (1 if r else 0)

    def body(tab_hbm, eidx_hbm, zeros_hbm, out_hbm,
             acc_sh, sidx_v, didx_v, *rest):
        rows = rest[:_NBUF]
        gsem = rest[_NBUF:2 * _NBUF]
        ssem = rest[2 * _NBUF:3 * _NBUF]
        zsem = rest[3 * _NBUF]
        c = lax.axis_index("c")
        s = lax.axis_index("s")
        w = c * _NS + s
        base = nb * w + jnp.minimum(w, r)
        z = pltpu.async_copy(zeros_hbm, acc_sh.at[pl.ds(s * rpt, rpt)], zsem)
        a = pltpu.async_copy(eidx_hbm.at[0, pl.ds(base, nb)],
                             sidx_v.at[pl.ds(0, nb)], gsem[0])
        b = pltpu.async_copy(eidx_hbm.at[1, pl.ds(base, nb)],
                             didx_v.at[pl.ds(0, nb)], gsem[1])
        z.wait()
        a.wait()
        b.wait()
        if r:
            def tail_stage():
                pltpu.sync_copy(eidx_hbm.at[0, pl.ds(base + nb, 1)],
                                sidx_v.at[pl.ds(nb, 1)])
                pltpu.sync_copy(eidx_hbm.at[1, pl.ds(base + nb, 1)],
                                didx_v.at[pl.ds(nb, 1)])
            pl.when(w < r)(tail_stage)
        plsc.subcore_barrier()

        def pipe(lo, hi):
            gd = [None] * _NBUF
            sd = [None] * _NBUF
            n_ = hi - lo
            for t in range(n_ + _DEP):
                if t < n_:
                    bi = t % _NBUF
                    if t >= _NBUF:
                        sd[bi].wait()
                    gd[bi] = pltpu.async_copy(
                        tab_hbm.at[sidx_v.at[lo + t]], rows[bi], gsem[bi])
                if t >= _DEP:
                    j = t - _DEP
                    bj = j % _NBUF
                    gd[bj].wait()
                    sd[bj] = pltpu.async_copy(
                        rows[bj], acc_sh.at[didx_v.at[lo + j]], ssem[bj],
                        add=True)
            for j in range(max(0, n_ - _NBUF), n_):
                sd[j % _NBUF].wait()

        pipe(0, nb)
        if r:
            def tail_chunk():
                pltpu.async_copy(tab_hbm.at[sidx_v.at[nb]], rows[0],
                                 gsem[0]).wait()
                pltpu.async_copy(rows[0], acc_sh.at[didx_v.at[nb]], ssem[0],
                                 add=True).wait()
            pl.when(w < r)(tail_chunk)
        plsc.subcore_barrier()
        pltpu.sync_copy(acc_sh.at[pl.ds(s * rpt, rpt)],
                        out_hbm.at[c, pl.ds(s * rpt, rpt)])

    f = pl.kernel(
        body,
        out_type=jax.ShapeDtypeStruct((_NC, npad, h), jnp.bfloat16),
        mesh=_sc_mesh(),
        scratch_types=(
            [pltpu.VMEM_SHARED((npad, h), jnp.bfloat16),
             pltpu.VMEM((kv, _B), jnp.int32),
             pltpu.VMEM((kv, _B), jnp.int32)]
            + [pltpu.VMEM((_B, h), jnp.bfloat16)] * _NBUF
            + [pltpu.SemaphoreType.DMA] * (2 * _NBUF + 1)
        ),
        compiler_params=_SC_PARAMS,
    )
    return f(table, eidx, zerosh)


def _dinv_of(degp_ref):
    deg = degp_ref[0, :, 0:1] + degp_ref[1, :, 0:1]
    return lax.rsqrt(deg + 1.0)


def _tc1(x, W1, degp, n, npad, h):
    def body(x_ref, w1_ref, degp_ref, out_ref):
        t1 = jnp.dot(x_ref[...], w1_ref[...], preferred_element_type=jnp.float32)
        dinv = _dinv_of(degp_ref)
        out_ref[pl.ds(0, n), :] = (t1 * dinv[:n]).astype(jnp.bfloat16)
        out_ref[pl.ds(n, npad - n), :] = jnp.zeros((npad - n, h), jnp.bfloat16)

    return pl.pallas_call(
        body, out_shape=jax.ShapeDtypeStruct((npad, h), jnp.bfloat16),
    )(x, W1, degp)


def _tc2(e1, h1p, degp, b1, npad, h):
    def body(e_ref, h1p_ref, degp_ref, b1_ref, out_ref):
        dinv = _dinv_of(degp_ref)
        agg = (e_ref[0].astype(jnp.float32) + e_ref[1].astype(jnp.float32)
               + h1p_ref[...].astype(jnp.float32)) * dinv
        h1 = jnp.maximum(agg + b1_ref[...], 0.0)
        out_ref[...] = (h1 * dinv).astype(jnp.bfloat16)

    return pl.pallas_call(
        body, out_shape=jax.ShapeDtypeStruct((npad, h), jnp.bfloat16),
    )(e1, h1p, degp, b1)


def _tc3(e2, v2, degp, W2, b2, batch2d, fc1_w, fc1_b, fc2_w, fc2_b, npad, ncls):
    def body(e_ref, v2_ref, degp_ref, w2_ref, b2_ref, batch_ref,
             fc1w_ref, fc1b_ref, fc2w_ref, fc2b_ref, out_ref):
        dinv = _dinv_of(degp_ref)
        a2 = (e_ref[0].astype(jnp.float32) + e_ref[1].astype(jnp.float32)
              + v2_ref[...].astype(jnp.float32)) * dinv
        h2 = jnp.maximum(
            jnp.dot(a2, w2_ref[...], preferred_element_type=jnp.float32)
            + b2_ref[...], 0.0)
        g = lax.broadcasted_iota(jnp.int32, (_G, 1), 0)
        mask = (batch_ref[...] == g).astype(jnp.float32)
        sums = jnp.dot(mask, h2, preferred_element_type=jnp.float32)
        counts = jnp.sum(mask, axis=1, keepdims=True)
        pooled = sums / jnp.maximum(counts, 1.0)
        z1 = jnp.maximum(
            jnp.dot(pooled, fc1w_ref[...], preferred_element_type=jnp.float32)
            + fc1b_ref[...], 0.0)
        out_ref[...] = (
            jnp.dot(z1, fc2w_ref[...], preferred_element_type=jnp.float32)
            + fc2b_ref[...])

    return pl.pallas_call(
        body, out_shape=jax.ShapeDtypeStruct((_G, ncls), jnp.float32),
    )(e2, v2, degp, W2, b2, batch2d, fc1_w, fc1_b, fc2_w, fc2_b)


def kernel(x, edge_index, batch, W1, b1, W2, b2, fc1_w, fc1_b, fc2_w, fc2_b):
    n, d_in = x.shape
    h = W1.shape[1]
    h2 = W2.shape[1]
    ncls = fc2_w.shape[1]
    e = edge_index.shape[1]

    nt = -(-e // _B)
    nb = nt // _NW
    r = nt % _NW
    rpt = -(-(n + 1) // _NS)
    rpt += (-rpt) % 8
    npad = rpt * _NS

    ei = edge_index.astype(jnp.int32)
    if e % _B:
        fill = jnp.full((2, nt * _B - e), n, jnp.int32)
        ei = jnp.concatenate([ei, fill], axis=1)
    eidx = ei.reshape(2, nt, _B)

    batch2d = jnp.pad(batch.astype(jnp.int32), (0, npad - n),
                      constant_values=_G).reshape(1, npad)

    ones8 = jnp.ones((_B, 8), jnp.float32)
    zeros8 = jnp.zeros((rpt, 8), jnp.float32)
    zerosh = jnp.zeros((rpt, h), jnp.bfloat16)

    degp = _deg_call(eidx, ones8, zeros8, npad, nb, r)
    h1p = _tc1(x, W1, degp, n, npad, h)
    e1 = _agg_call(h1p, eidx, zerosh, npad, nb, r, h)
    v2 = _tc2(e1, h1p, degp, b1.reshape(1, h), npad, h)
    e2 = _agg_call(v2, eidx, zerosh, npad, nb, r, h)
    out = _tc3(e2, v2, degp, W2, b2.reshape(1, h2), batch2d,
               fc1_w, fc1_b.reshape(1, h), fc2_w, fc2_b.reshape(1, ncls),
               npad, ncls)
    return out
